# baseline (speedup 1.0000x reference)
import functools

import jax
import jax.numpy as jnp
from jax import lax
from jax.experimental import pallas as pl
from jax.experimental.pallas import tpu as pltpu
from jax.experimental.pallas import tpu_sc as plsc

_NUM_S = 256
_NUM_T = 32
_D = 1024
_C = 16
_NB = 3


def _gather_add(spatio, temporal, pos_chunks, n_rows):
    info = plsc.get_sparse_core_info()
    nw = info.num_cores * info.num_subcores
    bpw = n_rows // nw
    n_chunks = pos_chunks.shape[1]

    mesh = plsc.VectorSubcoreMesh(core_axis_name="c", subcore_axis_name="s")

    @functools.partial(
        pl.kernel,
        mesh=mesh,
        out_type=jax.ShapeDtypeStruct((n_rows, _D), jnp.float32),
        scratch_types=[
            pltpu.VMEM((n_chunks, _C), jnp.int32),
            pltpu.VMEM((n_chunks, _C), jnp.int32),
            pltpu.VMEM((n_chunks, _C), jnp.int32),
            pltpu.VMEM((_NB, _C, _D), jnp.float32),
            pltpu.VMEM((_NB, _C, _D), jnp.float32),
            pltpu.SemaphoreType.DMA,
            pltpu.SemaphoreType.DMA,
            pltpu.SemaphoreType.DMA,
            pltpu.SemaphoreType.DMA,
            pltpu.SemaphoreType.DMA,
            pltpu.SemaphoreType.DMA,
            pltpu.SemaphoreType.DMA,
            pltpu.SemaphoreType.DMA,
            pltpu.SemaphoreType.DMA,
        ],
    )
    def k(sp_hbm, tm_hbm, pos_hbm, out_hbm, pos_v, sidx_v, tidx_v, sbuf, tbuf,
          sg0, sg1, sg2, tg0, tg1, tg2, os0, os1, os2):
        wid = lax.axis_index("s") * info.num_cores + lax.axis_index("c")
        base = wid * bpw
        pltpu.sync_copy(pos_hbm.at[wid], pos_v)

        sg, tg, osm = (sg0, sg1, sg2), (tg0, tg1, tg2), (os0, os1, os2)

        def split_idx(j, carry):
            p = pos_v[j, :]
            sidx_v[j, :] = jnp.bitwise_and(p, _NUM_S - 1)
            tidx_v[j, :] = jnp.right_shift(p, 8)
            return carry

        lax.fori_loop(0, n_chunks, split_idx, 0)

        def sdesc(j, b):
            return pltpu.make_async_copy(
                sp_hbm.at[sidx_v.at[j]], sbuf.at[b], sg[b]
            )

        def tdesc(j, b):
            return pltpu.make_async_copy(
                tm_hbm.at[tidx_v.at[j]], tbuf.at[b], tg[b]
            )

        def out_desc(j, b):
            return pltpu.make_async_copy(
                sbuf.at[b], out_hbm.at[pl.ds(base + j * _C, _C)], osm[b]
            )

        def start_gathers(j, b):
            sdesc(j, b).start()
            tdesc(j, b).start()

        def step(j, b):
            sdesc(j, b).wait()
            tdesc(j, b).wait()

            def add_row(r, carry):
                for kk in range(_D // 16):
                    d = pl.ds(kk * 16, 16)
                    plsc.addupdate(sbuf.at[b].at[r, d], tbuf[b, r, d])
                return carry

            lax.fori_loop(0, _C, add_row, 0)
            out_desc(j, b).start()

            @pl.when(j + 2 < n_chunks)
            def _():
                @pl.when(j >= 1)
                def _():
                    out_desc(j - 1, (b + 2) % _NB).wait()

                start_gathers(j + 2, (b + 2) % _NB)

        start_gathers(0, 0)
        start_gathers(1, 1)
        step(0, 0)

        n_main = (n_chunks - 2) // _NB * _NB

        def g_body(g, carry):
            for m in range(_NB):
                j = 1 + _NB * g + m
                step(j, (1 + m) % _NB)
            return carry

        lax.fori_loop(0, n_main // _NB, g_body, 0)
        for j in range(1 + n_main, n_chunks):
            step(j, j % _NB)
        for j in range(n_chunks - _NB, n_chunks):
            out_desc(j, j % _NB).wait()

    return k(spatio, temporal, pos_chunks)


def kernel(positions, spatio_table, temporal_table):
    n_rows = positions.size
    pos_chunks = positions.reshape(32, n_rows // (32 * _C), _C).astype(jnp.int32)
    out = _gather_add(spatio_table, temporal_table, pos_chunks, n_rows)
    return out.reshape(positions.shape + (_D,))


# speedup vs baseline: 2.3837x; 2.3837x over previous
import functools

import jax
import jax.numpy as jnp
from jax import lax
from jax.experimental import pallas as pl
from jax.experimental.pallas import tpu as pltpu
from jax.experimental.pallas import tpu_sc as plsc

_NUM_S = 256
_NUM_T = 32
_D = 1024


_TB = 8


def _build_body(spatio_ref, temporal_ref, out_ref):
    rows = temporal_ref[pl.ds(pl.program_id(0) * _TB, _TB), :]
    out_ref[...] = spatio_ref[...][None, :, :] + rows[:, None, :]


def _build_combined(spatio, temporal):
    out = pl.pallas_call(
        _build_body,
        grid=(_NUM_T // _TB,),
        in_specs=[
            pl.BlockSpec((_NUM_S, _D), lambda t: (0, 0)),
            pl.BlockSpec((_NUM_T, _D), lambda t: (0, 0)),
        ],
        out_specs=pl.BlockSpec((_TB, _NUM_S, _D), lambda t: (t, 0, 0)),
        out_shape=jax.ShapeDtypeStruct((_NUM_T, _NUM_S, _D), jnp.float32),
    )(spatio, temporal)
    return out.reshape(_NUM_T * _NUM_S, _D)


def _gather_rows(combined, positions, n_rows, c):
    info = plsc.get_sparse_core_info()
    nw = info.num_cores * info.num_subcores
    bpw = n_rows // nw
    n_chunks = bpw // c
    wpr = positions.shape[1] // bpw

    mesh = plsc.VectorSubcoreMesh(core_axis_name="c", subcore_axis_name="s")

    nb = 3

    @functools.partial(
        pl.kernel,
        mesh=mesh,
        out_type=jax.ShapeDtypeStruct((n_rows, _D), jnp.float32),
        scratch_types=[
            pltpu.VMEM((bpw,), jnp.int32),
            pltpu.VMEM((c, _D), jnp.float32),
            pltpu.VMEM((c, _D), jnp.float32),
            pltpu.VMEM((c, _D), jnp.float32),
            pltpu.SemaphoreType.DMA,
            pltpu.SemaphoreType.DMA,
            pltpu.SemaphoreType.DMA,
            pltpu.SemaphoreType.DMA,
            pltpu.SemaphoreType.DMA,
            pltpu.SemaphoreType.DMA,
        ],
    )
    def k(comb_hbm, pos_hbm, out_hbm, idx_v, b0, b1, b2, g0, g1, g2, o0, o1, o2):
        wid = lax.axis_index("s") * info.num_cores + lax.axis_index("c")
        base = wid * bpw
        pltpu.sync_copy(
            pos_hbm.at[wid // wpr, pl.ds((wid % wpr) * bpw, bpw)], idx_v
        )

        bufs, gs, osm = (b0, b1, b2), (g0, g1, g2), (o0, o1, o2)

        def gather_desc(j, b):
            return pltpu.make_async_copy(
                comb_hbm.at[idx_v.at[pl.ds(j * c, c)]], bufs[b], gs[b]
            )

        def out_desc(j, b):
            return pltpu.make_async_copy(
                bufs[b], out_hbm.at[pl.ds(base + j * c, c)], osm[b]
            )

        def step(j, b):
            gather_desc(j, b).wait()
            out_desc(j, b).start()

            @pl.when(j + 2 < n_chunks)
            def _():
                @pl.when(j >= 1)
                def _():
                    out_desc(j - 1, (b + 2) % nb).wait()

                gather_desc(j + 2, (b + 2) % nb).start()

        gather_desc(0, 0).start()
        gather_desc(1, 1).start()
        step(0, 0)

        def g_body(g, carry):
            for m in range(nb):
                j = 1 + nb * g + m
                step(j, (1 + m) % nb)
            return carry

        lax.fori_loop(0, (n_chunks - 2) // nb, g_body, 0)
        step(n_chunks - 1, (n_chunks - 1) % nb)
        for j in (n_chunks - 3, n_chunks - 2, n_chunks - 1):
            out_desc(j, j % nb).wait()

    return k(combined, positions)


def kernel(positions, spatio_table, temporal_table):
    combined = _build_combined(spatio_table, temporal_table)
    n_rows = positions.size
    c = 32
    out = _gather_rows(combined, positions.astype(jnp.int32), n_rows, c)
    return out.reshape(positions.shape + (_D,))
